# baseline (speedup 1.0000x reference)
import functools

import jax
import jax.numpy as jnp
from jax import lax
from jax.experimental import pallas as pl
from jax.experimental.pallas import tpu as pltpu
from jax.experimental.pallas import tpu_sc as plsc

NUM_CORES = 2
NUM_SUBCORES = 16
NUM_WORKERS = NUM_CORES * NUM_SUBCORES


def _iota16():
    return lax.iota(jnp.int32, 16)


def _rolls():
    return [(_iota16() + k) & 15 for k in range(16)]


@functools.lru_cache(maxsize=None)
def _build_pack(v: int, d: int):
    assert d == 64
    n_full = v // 128
    tail = v - n_full * 128
    assert tail % 2 == 0
    base_cnt = (n_full // NUM_WORKERS) & ~1
    n_extra = n_full - base_cnt * NUM_WORKERS
    n_pairs = base_cnt // 2
    mesh = plsc.VectorSubcoreMesh(core_axis_name="c", subcore_axis_name="s")

    @functools.partial(
        pl.kernel,
        mesh=mesh,
        out_type=jax.ShapeDtypeStruct((v // 2, 2 * d), jnp.float32),
        scratch_types=[
            pltpu.VMEM((d, 128), jnp.float32),
            pltpu.VMEM((d, 128), jnp.float32),
            pltpu.VMEM((d, 128), jnp.float32),
            pltpu.VMEM((d, 128), jnp.float32),
            pltpu.SemaphoreType.DMA,
            pltpu.SemaphoreType.DMA,
            pltpu.SemaphoreType.DMA,
            pltpu.SemaphoreType.DMA,
        ],
        compiler_params=pltpu.CompilerParams(needs_layout_passes=False),
    )
    def pack_kernel(tt_hbm, rt_hbm, src_a, src_b, dst_a, dst_b,
                    sin_a, sin_b, sout_a, sout_b):
        wid = lax.axis_index("s") * NUM_CORES + lax.axis_index("c")
        rolls = _rolls()

        def start_load(rt, buf, sem):
            for dt in range(d // 8):
                pltpu.async_copy(
                    tt_hbm.at[pl.ds(dt * 8, 8), pl.ds(rt * 128, 128)],
                    buf.at[pl.ds(dt * 8, 8), :],
                    sem,
                )

        def drain_load(buf, sem):
            for dt in range(d // 8):
                pltpu.make_async_copy(
                    tt_hbm.at[pl.ds(0, 8), pl.ds(0, 128)],
                    buf.at[pl.ds(dt * 8, 8), :],
                    sem,
                ).wait()

        def transpose(src, dst, n_c_blocks):
            @plsc.parallel_loop(0, n_c_blocks * (d // 16), unroll=4)
            def cbody(i):
                c0 = lax.shift_right_logical(i, 2)
                d0 = i & 3
                c_vec = c0 * 16 + _iota16()
                qrow = lax.shift_right_logical(c_vec, 1)
                qcol = lax.shift_left(c_vec & 1, 6)
                for k in range(16):
                    d_vec = d0 * 16 + rolls[k]
                    val = plsc.load_gather(src, [d_vec, c_vec])
                    plsc.store_scatter(dst, [qrow, qcol + d_vec], val)

        def store(rt, buf, nrows, sem):
            pltpu.async_copy(
                buf.at[pl.ds(0, nrows), :],
                rt_hbm.at[pl.ds(rt * 64, nrows), :],
                sem,
            )

        def drain_store(buf, nrows, sem):
            pltpu.make_async_copy(
                tt_hbm.at[pl.ds(0, nrows), pl.ds(0, 128)],
                buf.at[pl.ds(0, nrows), :],
                sem,
            ).wait()

        start_load(wid, src_a, sin_a)
        start_load(wid + NUM_WORKERS, src_b, sin_b)

        def pair_body(p, carry):
            for slot, (src, dst, sin, sout) in enumerate(
                ((src_a, dst_a, sin_a, sout_a), (src_b, dst_b, sin_b, sout_b))
            ):
                k = 2 * p + slot
                drain_load(src, sin)

                @pl.when(p > 0)
                def _():
                    drain_store(dst, 64, sout)

                transpose(src, dst, 8)

                @pl.when(k + 2 < base_cnt)
                def _():
                    start_load(wid + (k + 2) * NUM_WORKERS, src, sin)

                store(wid + k * NUM_WORKERS, dst, 64, sout)
            return carry

        lax.fori_loop(0, n_pairs, pair_body, 0)
        drain_store(dst_a, 64, sout_a)
        drain_store(dst_b, 64, sout_b)

        @pl.when(wid < n_extra)
        def _():
            rt = base_cnt * NUM_WORKERS + wid
            start_load(rt, src_a, sin_a)
            drain_load(src_a, sin_a)
            transpose(src_a, dst_a, 8)
            store(rt, dst_a, 64, sout_a)
            drain_store(dst_a, 64, sout_a)

        if tail:
            @pl.when(wid == n_extra)
            def _():
                for row in range(d):
                    pltpu.async_copy(
                        tt_hbm.at[row, pl.ds(n_full * 128, tail)],
                        src_a.at[row, pl.ds(0, tail)],
                        sin_a,
                    )
                for row in range(d):
                    pltpu.make_async_copy(
                        tt_hbm.at[row, pl.ds(0, tail)],
                        src_a.at[row, pl.ds(0, tail)],
                        sin_a,
                    ).wait()
                transpose(src_a, dst_a, tail // 16)
                store(n_full, dst_a, tail // 2, sout_a)
                drain_store(dst_a, tail // 2, sout_a)

    return pack_kernel


@functools.lru_cache(maxsize=None)
def _build_gather(v: int, d: int, n_h: int, n_b: int):
    assert d == 64 and n_h % 8 == 0 and n_b % 128 == 0
    n_units = (n_h // 8) * (n_b // 128)
    assert n_units % NUM_WORKERS == 0
    upw = n_units // NUM_WORKERS
    nbt = n_b // 128
    mesh = plsc.VectorSubcoreMesh(core_axis_name="c", subcore_axis_name="s")

    @functools.partial(
        pl.kernel,
        mesh=mesh,
        out_type=jax.ShapeDtypeStruct((n_h, d, n_b), jnp.float32),
        scratch_types=[
            pltpu.VMEM((8, 128), jnp.int32),
            pltpu.VMEM((8, 128), jnp.int32),
            pltpu.VMEM((128, 128), jnp.float32),
            pltpu.VMEM((128, 128), jnp.float32),
            pltpu.VMEM((128, 128), jnp.float32),
            pltpu.VMEM((d, 128), jnp.float32),
            pltpu.VMEM((d, 128), jnp.float32),
            pltpu.SemaphoreType.DMA,
            pltpu.SemaphoreType.DMA,
            pltpu.SemaphoreType.DMA,
            pltpu.SemaphoreType.DMA,
            pltpu.SemaphoreType.DMA,
        ],
        compiler_params=pltpu.CompilerParams(needs_layout_passes=False),
    )
    def gather_kernel(rt_hbm, xt_hbm, ot_hbm, idxt, qt, gb_a, gb_b, gb_c,
                      ob_a, ob_b, sg_a, sg_b, sg_c, so_a, so_b):
        wid = lax.axis_index("s") * NUM_CORES + lax.axis_index("c")
        rolls = _rolls()
        gbufs = (gb_a, gb_b, gb_c)
        gsems = (sg_a, sg_b, sg_c)
        obufs = (ob_a, ob_b)
        osems = (so_a, so_b)

        def gather(hh, buf, sem):
            pltpu.async_copy(rt_hbm.at[qt.at[hh]], buf, sem)

        def drain_gather(buf, sem):
            pltpu.make_async_copy(rt_hbm.at[pl.ds(0, 128), :], buf, sem).wait()

        def transpose(hh, gbuf, obuf):
            @plsc.parallel_loop(0, 8 * (d // 16), unroll=4)
            def bbody(i):
                b0 = lax.shift_right_logical(i, 2)
                d0 = i & 3
                b_vec = b0 * 16 + _iota16()
                par = lax.shift_left(idxt[hh, pl.ds(b0 * 16, 16)] & 1, 6)
                for k in range(16):
                    d_vec = d0 * 16 + rolls[k]
                    val = plsc.load_gather(gbuf, [b_vec, par + d_vec])
                    plsc.store_scatter(obuf, [d_vec, b_vec], val)

        def store_out(h, bb, obuf, sem):
            for dt in range(d // 8):
                pltpu.async_copy(
                    obuf.at[pl.ds(dt * 8, 8), :],
                    ot_hbm.at[h, pl.ds(dt * 8, 8), pl.ds(bb * 128, 128)],
                    sem,
                )

        def drain_out(obuf, sem):
            for dt in range(d // 8):
                pltpu.make_async_copy(
                    rt_hbm.at[pl.ds(0, 8), :],
                    obuf.at[pl.ds(dt * 8, 8), :],
                    sem,
                ).wait()

        def unit_body(t, carry):
            u = wid * upw + t
            ht = u // nbt
            bb = u % nbt
            pltpu.sync_copy(
                xt_hbm.at[pl.ds(ht * 8, 8), pl.ds(bb * 128, 128)], idxt
            )

            @plsc.parallel_loop(0, 8)
            def qbody(r):
                for g in range(8):
                    qt[r, pl.ds(g * 16, 16)] = idxt[r, pl.ds(g * 16, 16)] >> 1

            for hh in range(3):
                gather(hh, gbufs[hh], gsems[hh])

            for hh in range(8):
                gbuf, sg = gbufs[hh % 3], gsems[hh % 3]
                obuf, so = obufs[hh % 2], osems[hh % 2]
                drain_gather(gbuf, sg)
                if hh < 2:
                    @pl.when(t > 0)
                    def _():
                        drain_out(obuf, so)
                else:
                    drain_out(obuf, so)
                transpose(hh, gbuf, obuf)
                if hh + 3 < 8:
                    gather(hh + 3, gbuf, sg)
                store_out(ht * 8 + hh, bb, obuf, so)
            return carry

        lax.fori_loop(0, upw, unit_body, 0)
        drain_out(ob_a, so_a)
        drain_out(ob_b, so_b)

    return gather_kernel


def kernel(x, table):
    batch, hist = x.shape
    v, d = table.shape
    table_t = table.T
    x_t = x.T
    rt = _build_pack(v, d)(table_t)
    ot = _build_gather(v, d, hist, batch)(rt, x_t)
    return jnp.transpose(ot, (2, 0, 1))


# speedup vs baseline: 1.4783x; 1.4783x over previous
import functools

import jax
import jax.numpy as jnp
from jax import lax
from jax.experimental import pallas as pl
from jax.experimental.pallas import tpu as pltpu
from jax.experimental.pallas import tpu_sc as plsc

NUM_CORES = 2
NUM_SUBCORES = 16
NUM_WORKERS = NUM_CORES * NUM_SUBCORES


def _iota16():
    return lax.iota(jnp.int32, 16)


def _rolls():
    return [(_iota16() + k) & 15 for k in range(16)]


@functools.lru_cache(maxsize=None)
def _build_pack(v: int, d: int):
    assert d == 64
    n_full = v // 128
    tail = v - n_full * 128
    assert tail % 2 == 0
    base_cnt = (n_full // NUM_WORKERS) & ~1
    n_extra = n_full - base_cnt * NUM_WORKERS
    n_pairs = base_cnt // 2
    mesh = plsc.VectorSubcoreMesh(core_axis_name="c", subcore_axis_name="s")

    @functools.partial(
        pl.kernel,
        mesh=mesh,
        out_type=jax.ShapeDtypeStruct((v // 2, 2 * d), jnp.float32),
        scratch_types=[
            pltpu.VMEM((d, 128), jnp.float32),
            pltpu.VMEM((d, 128), jnp.float32),
            pltpu.VMEM((d, 128), jnp.float32),
            pltpu.VMEM((d, 128), jnp.float32),
            pltpu.SemaphoreType.DMA,
            pltpu.SemaphoreType.DMA,
            pltpu.SemaphoreType.DMA,
            pltpu.SemaphoreType.DMA,
        ],
        compiler_params=pltpu.CompilerParams(needs_layout_passes=False),
    )
    def pack_kernel(tt_hbm, rt_hbm, src_a, src_b, dst_a, dst_b,
                    sin_a, sin_b, sout_a, sout_b):
        wid = lax.axis_index("s") * NUM_CORES + lax.axis_index("c")
        rolls = _rolls()

        def start_load(rt, buf, sem):
            for dt in range(d // 8):
                pltpu.async_copy(
                    tt_hbm.at[pl.ds(dt * 8, 8), pl.ds(rt * 128, 128)],
                    buf.at[pl.ds(dt * 8, 8), :],
                    sem,
                )

        def drain_load(buf, sem):
            for dt in range(d // 8):
                pltpu.make_async_copy(
                    tt_hbm.at[pl.ds(0, 8), pl.ds(0, 128)],
                    buf.at[pl.ds(dt * 8, 8), :],
                    sem,
                ).wait()

        def transpose(src, dst, n_c_blocks):
            @plsc.parallel_loop(0, n_c_blocks * (d // 16), unroll=2)
            def cbody(i):
                c0 = lax.shift_right_logical(i, 2)
                d0 = i & 3
                c_vec = c0 * 16 + _iota16()
                qrow = lax.shift_right_logical(c_vec, 1)
                qcol = lax.shift_left(c_vec & 1, 6)
                for k in range(16):
                    d_vec = d0 * 16 + rolls[k]
                    val = plsc.load_gather(src, [d_vec, c_vec])
                    plsc.store_scatter(dst, [qrow, qcol + d_vec], val)

        def store(rt, buf, nrows, sem):
            pltpu.async_copy(
                buf.at[pl.ds(0, nrows), :],
                rt_hbm.at[pl.ds(rt * 64, nrows), :],
                sem,
            )

        def drain_store(buf, nrows, sem):
            pltpu.make_async_copy(
                tt_hbm.at[pl.ds(0, nrows), pl.ds(0, 128)],
                buf.at[pl.ds(0, nrows), :],
                sem,
            ).wait()

        start_load(wid, src_a, sin_a)
        start_load(wid + NUM_WORKERS, src_b, sin_b)

        def pair_body(p, carry):
            for slot, (src, dst, sin, sout) in enumerate(
                ((src_a, dst_a, sin_a, sout_a), (src_b, dst_b, sin_b, sout_b))
            ):
                k = 2 * p + slot
                drain_load(src, sin)

                @pl.when(p > 0)
                def _():
                    drain_store(dst, 64, sout)

                transpose(src, dst, 8)

                @pl.when(k + 2 < base_cnt)
                def _():
                    start_load(wid + (k + 2) * NUM_WORKERS, src, sin)

                store(wid + k * NUM_WORKERS, dst, 64, sout)
            return carry

        lax.fori_loop(0, n_pairs, pair_body, 0)
        drain_store(dst_a, 64, sout_a)
        drain_store(dst_b, 64, sout_b)

        @pl.when(wid < n_extra)
        def _():
            rt = base_cnt * NUM_WORKERS + wid
            start_load(rt, src_a, sin_a)
            drain_load(src_a, sin_a)
            transpose(src_a, dst_a, 8)
            store(rt, dst_a, 64, sout_a)
            drain_store(dst_a, 64, sout_a)

        if tail:
            @pl.when(wid == n_extra)
            def _():
                for row in range(d):
                    pltpu.async_copy(
                        tt_hbm.at[row, pl.ds(n_full * 128, tail)],
                        src_a.at[row, pl.ds(0, tail)],
                        sin_a,
                    )
                for row in range(d):
                    pltpu.make_async_copy(
                        tt_hbm.at[row, pl.ds(0, tail)],
                        src_a.at[row, pl.ds(0, tail)],
                        sin_a,
                    ).wait()
                transpose(src_a, dst_a, tail // 16)
                store(n_full, dst_a, tail // 2, sout_a)
                drain_store(dst_a, tail // 2, sout_a)

    return pack_kernel


@functools.lru_cache(maxsize=None)
def _build_gather(v: int, d: int, n_h: int, n_b: int):
    assert d == 64 and n_h % 8 == 0 and n_b % 128 == 0
    n_units = (n_h // 8) * (n_b // 128)
    assert n_units % NUM_WORKERS == 0
    upw = n_units // NUM_WORKERS
    nbt = n_b // 128
    mesh = plsc.VectorSubcoreMesh(core_axis_name="c", subcore_axis_name="s")

    @functools.partial(
        pl.kernel,
        mesh=mesh,
        out_type=jax.ShapeDtypeStruct((n_h, d, n_b), jnp.float32),
        scratch_types=[
            pltpu.VMEM((8, 128), jnp.int32),
            pltpu.VMEM((8, 128), jnp.int32),
            pltpu.VMEM((128, 128), jnp.float32),
            pltpu.VMEM((128, 128), jnp.float32),
            pltpu.VMEM((128, 128), jnp.float32),
            pltpu.VMEM((d, 128), jnp.float32),
            pltpu.VMEM((d, 128), jnp.float32),
            pltpu.SemaphoreType.DMA,
            pltpu.SemaphoreType.DMA,
            pltpu.SemaphoreType.DMA,
            pltpu.SemaphoreType.DMA,
            pltpu.SemaphoreType.DMA,
        ],
        compiler_params=pltpu.CompilerParams(needs_layout_passes=False),
    )
    def gather_kernel(rt_hbm, xt_hbm, ot_hbm, idxt, qt, gb_a, gb_b, gb_c,
                      ob_a, ob_b, sg_a, sg_b, sg_c, so_a, so_b):
        wid = lax.axis_index("s") * NUM_CORES + lax.axis_index("c")
        rolls = _rolls()
        gbufs = (gb_a, gb_b, gb_c)
        gsems = (sg_a, sg_b, sg_c)
        obufs = (ob_a, ob_b)
        osems = (so_a, so_b)

        def gather(hh, buf, sem):
            pltpu.async_copy(rt_hbm.at[qt.at[hh]], buf, sem)

        def drain_gather(buf, sem):
            pltpu.make_async_copy(rt_hbm.at[pl.ds(0, 128), :], buf, sem).wait()

        def transpose(hh, gbuf, obuf):
            @plsc.parallel_loop(0, 8 * (d // 16), unroll=2)
            def bbody(i):
                b0 = lax.shift_right_logical(i, 2)
                d0 = i & 3
                b_vec = b0 * 16 + _iota16()
                par = lax.shift_left(idxt[hh, pl.ds(b0 * 16, 16)] & 1, 6)
                for k in range(16):
                    d_vec = d0 * 16 + rolls[k]
                    val = plsc.load_gather(gbuf, [b_vec, par + d_vec])
                    plsc.store_scatter(obuf, [d_vec, b_vec], val)

        def store_out(h, bb, obuf, sem):
            for dt in range(d // 8):
                pltpu.async_copy(
                    obuf.at[pl.ds(dt * 8, 8), :],
                    ot_hbm.at[h, pl.ds(dt * 8, 8), pl.ds(bb * 128, 128)],
                    sem,
                )

        def drain_out(obuf, sem):
            for dt in range(d // 8):
                pltpu.make_async_copy(
                    rt_hbm.at[pl.ds(0, 8), :],
                    obuf.at[pl.ds(dt * 8, 8), :],
                    sem,
                ).wait()

        def unit_body(t, carry):
            u = wid * upw + t
            ht = u // nbt
            bb = u % nbt
            pltpu.sync_copy(
                xt_hbm.at[pl.ds(ht * 8, 8), pl.ds(bb * 128, 128)], idxt
            )

            @plsc.parallel_loop(0, 8)
            def qbody(r):
                for g in range(8):
                    qt[r, pl.ds(g * 16, 16)] = idxt[r, pl.ds(g * 16, 16)] >> 1

            for hh in range(3):
                gather(hh, gbufs[hh], gsems[hh])

            for hh in range(8):
                gbuf, sg = gbufs[hh % 3], gsems[hh % 3]
                obuf, so = obufs[hh % 2], osems[hh % 2]
                drain_gather(gbuf, sg)
                if hh < 2:
                    @pl.when(t > 0)
                    def _():
                        drain_out(obuf, so)
                else:
                    drain_out(obuf, so)
                transpose(hh, gbuf, obuf)
                if hh + 3 < 8:
                    gather(hh + 3, gbuf, sg)
                store_out(ht * 8 + hh, bb, obuf, so)
            return carry

        lax.fori_loop(0, upw, unit_body, 0)
        drain_out(ob_a, so_a)
        drain_out(ob_b, so_b)

    return gather_kernel


def kernel(x, table):
    batch, hist = x.shape
    v, d = table.shape
    table_t = table.T
    x_t = x.T
    rt = _build_pack(v, d)(table_t)
    ot = _build_gather(v, d, hist, batch)(rt, x_t)
    return jnp.transpose(ot, (2, 0, 1))


# speedup vs baseline: 1.4802x; 1.0013x over previous
import functools

import jax
import jax.numpy as jnp
from jax import lax
from jax.experimental import pallas as pl
from jax.experimental.pallas import tpu as pltpu
from jax.experimental.pallas import tpu_sc as plsc

NUM_CORES = 2
NUM_SUBCORES = 16
NUM_WORKERS = NUM_CORES * NUM_SUBCORES


def _iota16():
    return lax.iota(jnp.int32, 16)


def _rolls():
    return [(_iota16() + k) & 15 for k in range(16)]


@functools.lru_cache(maxsize=None)
def _build_pack(v: int, d: int):
    assert d == 64
    n_full = v // 128
    tail = v - n_full * 128
    assert tail % 2 == 0
    base_cnt = (n_full // NUM_WORKERS) & ~1
    n_extra = n_full - base_cnt * NUM_WORKERS
    n_pairs = base_cnt // 2
    mesh = plsc.VectorSubcoreMesh(core_axis_name="c", subcore_axis_name="s")

    @functools.partial(
        pl.kernel,
        mesh=mesh,
        out_type=jax.ShapeDtypeStruct((v // 2, 2 * d), jnp.float32),
        scratch_types=[
            pltpu.VMEM((d, 128), jnp.float32),
            pltpu.VMEM((d, 128), jnp.float32),
            pltpu.VMEM((d, 128), jnp.float32),
            pltpu.VMEM((d, 128), jnp.float32),
            pltpu.SemaphoreType.DMA,
            pltpu.SemaphoreType.DMA,
            pltpu.SemaphoreType.DMA,
            pltpu.SemaphoreType.DMA,
        ],
        compiler_params=pltpu.CompilerParams(needs_layout_passes=False),
    )
    def pack_kernel(tt_hbm, rt_hbm, src_a, src_b, dst_a, dst_b,
                    sin_a, sin_b, sout_a, sout_b):
        wid = lax.axis_index("s") * NUM_CORES + lax.axis_index("c")
        rolls = _rolls()

        def start_load(rt, buf, sem):
            pltpu.async_copy(
                tt_hbm.at[:, pl.ds(rt * 128, 128)],
                buf,
                sem,
            )

        def drain_load(buf, sem):
            pltpu.make_async_copy(
                tt_hbm.at[:, pl.ds(0, 128)],
                buf,
                sem,
            ).wait()

        def transpose(src, dst, n_c_blocks):
            @plsc.parallel_loop(0, n_c_blocks * (d // 16), unroll=2)
            def cbody(i):
                c0 = lax.shift_right_logical(i, 2)
                d0 = i & 3
                c_vec = c0 * 16 + _iota16()
                qrow = lax.shift_right_logical(c_vec, 1)
                qcol = lax.shift_left(c_vec & 1, 6)
                for k in range(16):
                    d_vec = d0 * 16 + rolls[k]
                    val = plsc.load_gather(src, [d_vec, c_vec])
                    plsc.store_scatter(dst, [qrow, qcol + d_vec], val)

        def store(rt, buf, nrows, sem):
            pltpu.async_copy(
                buf.at[pl.ds(0, nrows), :],
                rt_hbm.at[pl.ds(rt * 64, nrows), :],
                sem,
            )

        def drain_store(buf, nrows, sem):
            pltpu.make_async_copy(
                tt_hbm.at[pl.ds(0, nrows), pl.ds(0, 128)],
                buf.at[pl.ds(0, nrows), :],
                sem,
            ).wait()

        start_load(wid, src_a, sin_a)
        start_load(wid + NUM_WORKERS, src_b, sin_b)

        def pair_body(p, carry):
            for slot, (src, dst, sin, sout) in enumerate(
                ((src_a, dst_a, sin_a, sout_a), (src_b, dst_b, sin_b, sout_b))
            ):
                k = 2 * p + slot
                drain_load(src, sin)

                @pl.when(p > 0)
                def _():
                    drain_store(dst, 64, sout)

                transpose(src, dst, 8)

                @pl.when(k + 2 < base_cnt)
                def _():
                    start_load(wid + (k + 2) * NUM_WORKERS, src, sin)

                store(wid + k * NUM_WORKERS, dst, 64, sout)
            return carry

        lax.fori_loop(0, n_pairs, pair_body, 0)
        drain_store(dst_a, 64, sout_a)
        drain_store(dst_b, 64, sout_b)

        @pl.when(wid < n_extra)
        def _():
            rt = base_cnt * NUM_WORKERS + wid
            start_load(rt, src_a, sin_a)
            drain_load(src_a, sin_a)
            transpose(src_a, dst_a, 8)
            store(rt, dst_a, 64, sout_a)
            drain_store(dst_a, 64, sout_a)

        if tail:
            @pl.when(wid == n_extra)
            def _():
                for row in range(d):
                    pltpu.async_copy(
                        tt_hbm.at[row, pl.ds(n_full * 128, tail)],
                        src_a.at[row, pl.ds(0, tail)],
                        sin_a,
                    )
                for row in range(d):
                    pltpu.make_async_copy(
                        tt_hbm.at[row, pl.ds(0, tail)],
                        src_a.at[row, pl.ds(0, tail)],
                        sin_a,
                    ).wait()
                transpose(src_a, dst_a, tail // 16)
                store(n_full, dst_a, tail // 2, sout_a)
                drain_store(dst_a, tail // 2, sout_a)

    return pack_kernel


@functools.lru_cache(maxsize=None)
def _build_gather(v: int, d: int, n_h: int, n_b: int):
    assert d == 64 and n_h % 8 == 0 and n_b % 128 == 0
    n_units = (n_h // 8) * (n_b // 128)
    assert n_units % NUM_WORKERS == 0
    upw = n_units // NUM_WORKERS
    nbt = n_b // 128
    mesh = plsc.VectorSubcoreMesh(core_axis_name="c", subcore_axis_name="s")

    @functools.partial(
        pl.kernel,
        mesh=mesh,
        out_type=jax.ShapeDtypeStruct((n_h, d, n_b), jnp.float32),
        scratch_types=[
            pltpu.VMEM((8, 128), jnp.int32),
            pltpu.VMEM((8, 128), jnp.int32),
            pltpu.VMEM((128, 128), jnp.float32),
            pltpu.VMEM((128, 128), jnp.float32),
            pltpu.VMEM((128, 128), jnp.float32),
            pltpu.VMEM((d, 128), jnp.float32),
            pltpu.VMEM((d, 128), jnp.float32),
            pltpu.SemaphoreType.DMA,
            pltpu.SemaphoreType.DMA,
            pltpu.SemaphoreType.DMA,
            pltpu.SemaphoreType.DMA,
            pltpu.SemaphoreType.DMA,
        ],
        compiler_params=pltpu.CompilerParams(needs_layout_passes=False),
    )
    def gather_kernel(rt_hbm, xt_hbm, ot_hbm, idxt, qt, gb_a, gb_b, gb_c,
                      ob_a, ob_b, sg_a, sg_b, sg_c, so_a, so_b):
        wid = lax.axis_index("s") * NUM_CORES + lax.axis_index("c")
        rolls = _rolls()
        gbufs = (gb_a, gb_b, gb_c)
        gsems = (sg_a, sg_b, sg_c)
        obufs = (ob_a, ob_b)
        osems = (so_a, so_b)

        def gather(hh, buf, sem):
            pltpu.async_copy(rt_hbm.at[qt.at[hh]], buf, sem)

        def drain_gather(buf, sem):
            pltpu.make_async_copy(rt_hbm.at[pl.ds(0, 128), :], buf, sem).wait()

        def transpose(hh, gbuf, obuf):
            @plsc.parallel_loop(0, 8 * (d // 16), unroll=2)
            def bbody(i):
                b0 = lax.shift_right_logical(i, 2)
                d0 = i & 3
                b_vec = b0 * 16 + _iota16()
                par = lax.shift_left(idxt[hh, pl.ds(b0 * 16, 16)] & 1, 6)
                for k in range(16):
                    d_vec = d0 * 16 + rolls[k]
                    val = plsc.load_gather(gbuf, [b_vec, par + d_vec])
                    plsc.store_scatter(obuf, [d_vec, b_vec], val)

        def store_out(h, bb, obuf, sem):
            for dt in range(d // 8):
                pltpu.async_copy(
                    obuf.at[pl.ds(dt * 8, 8), :],
                    ot_hbm.at[h, pl.ds(dt * 8, 8), pl.ds(bb * 128, 128)],
                    sem,
                )

        def drain_out(obuf, sem):
            for dt in range(d // 8):
                pltpu.make_async_copy(
                    rt_hbm.at[pl.ds(0, 8), :],
                    obuf.at[pl.ds(dt * 8, 8), :],
                    sem,
                ).wait()

        def unit_body(t, carry):
            u = wid * upw + t
            ht = u // nbt
            bb = u % nbt
            pltpu.sync_copy(
                xt_hbm.at[pl.ds(ht * 8, 8), pl.ds(bb * 128, 128)], idxt
            )

            @plsc.parallel_loop(0, 8)
            def qbody(r):
                for g in range(8):
                    qt[r, pl.ds(g * 16, 16)] = idxt[r, pl.ds(g * 16, 16)] >> 1

            for hh in range(3):
                gather(hh, gbufs[hh], gsems[hh])

            for hh in range(8):
                gbuf, sg = gbufs[hh % 3], gsems[hh % 3]
                obuf, so = obufs[hh % 2], osems[hh % 2]
                drain_gather(gbuf, sg)
                if hh < 2:
                    @pl.when(t > 0)
                    def _():
                        drain_out(obuf, so)
                else:
                    drain_out(obuf, so)
                transpose(hh, gbuf, obuf)
                if hh + 3 < 8:
                    gather(hh + 3, gbuf, sg)
                store_out(ht * 8 + hh, bb, obuf, so)
            return carry

        lax.fori_loop(0, upw, unit_body, 0)
        drain_out(ob_a, so_a)
        drain_out(ob_b, so_b)

    return gather_kernel


def kernel(x, table):
    batch, hist = x.shape
    v, d = table.shape
    table_t = table.T
    x_t = x.T
    rt = _build_pack(v, d)(table_t)
    ot = _build_gather(v, d, hist, batch)(rt, x_t)
    return jnp.transpose(ot, (2, 0, 1))
